# Initial kernel scaffold; baseline (speedup 1.0000x reference)
import jax
import jax.numpy as jnp
from jax.experimental import pallas as pl


def kernel(x, edge_index, W0, b0, W1, b1, W2, b2):
    raise NotImplementedError("write your pallas kernel here")


# speedup vs baseline: 6.1181x; 6.1181x over previous
import functools

import jax
import jax.numpy as jnp
from jax import lax
from jax.experimental import pallas as pl
from jax.experimental.pallas import tpu as pltpu
from jax.experimental.pallas import tpu_sc as plsc

_N = 10000
_E = 320000
_D = 128
_NC = 2
_NS = 16
_NW = _NC * _NS
_CB = 128
_NCH = 80
_EPAD = _NW * _NCH * _CB
_IB = 8
_NIB = _NCH // _IB
_NPAD = 10240
_RPS = _NPAD // _NS
_RB = 1024
_GRID = _NPAD // _RB


def _sc_mesh():
    return plsc.VectorSubcoreMesh(
        core_axis_name="c", subcore_axis_name="s", num_cores=_NC, num_subcores=_NS
    )


@functools.partial(
    pl.kernel,
    out_type=jax.ShapeDtypeStruct((_NC, _NPAD, _D), jnp.float32),
    mesh=_sc_mesh(),
    scratch_types=[
        pltpu.VMEM((_NCH, _CB), jnp.int32),
        pltpu.VMEM((_CB, _D), jnp.float32),
        pltpu.VMEM_SHARED((_NPAD, _D), jnp.float32),
    ],
)
def _deg_sc(dst_hbm, ones_hbm, zeros_hbm, out_hbm, dst_v, ones_v, acc_s):
    c = lax.axis_index("c")
    s = lax.axis_index("s")
    wid = s * _NC + c
    pltpu.sync_copy(zeros_hbm, acc_s.at[pl.ds(s * _RPS, _RPS)])
    pltpu.sync_copy(dst_hbm.at[wid], dst_v)
    pltpu.sync_copy(ones_hbm, ones_v)
    plsc.subcore_barrier()

    def body(i, carry):
        pltpu.sync_copy(ones_v, acc_s.at[dst_v.at[i]], add=True)
        return carry

    lax.fori_loop(0, _NCH, body, 0)
    plsc.subcore_barrier()
    pltpu.sync_copy(acc_s.at[pl.ds(s * _RPS, _RPS)],
                    out_hbm.at[c, pl.ds(s * _RPS, _RPS)])


@functools.partial(
    pl.kernel,
    out_type=jax.ShapeDtypeStruct((_NC, _NPAD, _D), jnp.float32),
    mesh=_sc_mesh(),
    scratch_types=[
        pltpu.VMEM((_IB, _CB), jnp.int32),
        pltpu.VMEM((_IB, _CB), jnp.int32),
        pltpu.VMEM((2, _CB, _D), jnp.float32),
        pltpu.VMEM_SHARED((_NPAD, _D), jnp.float32),
        pltpu.SemaphoreType.DMA,
        pltpu.SemaphoreType.DMA,
    ],
)
def _agg_sc(hp_hbm, src_hbm, dst_hbm, zeros_hbm, out_hbm,
            src_v, dst_v, rows_v, acc_s, sem0, sem1):
    c = lax.axis_index("c")
    s = lax.axis_index("s")
    wid = s * _NC + c
    pltpu.sync_copy(zeros_hbm, acc_s.at[pl.ds(s * _RPS, _RPS)])
    plsc.subcore_barrier()

    def batch(ib, carry):
        pltpu.sync_copy(src_hbm.at[wid, pl.ds(ib * _IB, _IB)], src_v)
        pltpu.sync_copy(dst_hbm.at[wid, pl.ds(ib * _IB, _IB)], dst_v)

        def pair(jp, carry2):
            a = 2 * jp
            ga = pltpu.async_copy(hp_hbm.at[src_v.at[a]], rows_v.at[0], sem0)
            gb = pltpu.async_copy(hp_hbm.at[src_v.at[a + 1]], rows_v.at[1], sem1)
            ga.wait()
            pltpu.sync_copy(rows_v.at[0], acc_s.at[dst_v.at[a]], add=True)
            gb.wait()
            pltpu.sync_copy(rows_v.at[1], acc_s.at[dst_v.at[a + 1]], add=True)
            return carry2

        lax.fori_loop(0, _IB // 2, pair, 0)
        return carry

    lax.fori_loop(0, _NIB, batch, 0)

    plsc.subcore_barrier()
    pltpu.sync_copy(acc_s.at[pl.ds(s * _RPS, _RPS)],
                    out_hbm.at[c, pl.ds(s * _RPS, _RPS)])


def _prelude_body(deg_ref, x_ref, w_ref, dis_ref, hp_ref):
    deg = deg_ref[:, 0:1] + deg_ref[:, 1:2] + 1.0
    dis = lax.rsqrt(deg)
    dis_ref[...] = dis
    h = jnp.dot(x_ref[...], w_ref[...], preferred_element_type=jnp.float32)
    hp_ref[...] = h * dis


_prelude_tc = pl.pallas_call(
    _prelude_body,
    grid=(_GRID,),
    in_specs=[
        pl.BlockSpec((_RB, 2), lambda i: (i, 0)),
        pl.BlockSpec((_RB, _D), lambda i: (i, 0)),
        pl.BlockSpec((_D, _D), lambda i: (0, 0)),
    ],
    out_specs=[
        pl.BlockSpec((_RB, 1), lambda i: (i, 0)),
        pl.BlockSpec((_RB, _D), lambda i: (i, 0)),
    ],
    out_shape=[
        jax.ShapeDtypeStruct((_NPAD, 1), jnp.float32),
        jax.ShapeDtypeStruct((_NPAD, _D), jnp.float32),
    ],
)


def _mid_body(s2_ref, hp_ref, dis_ref, w_ref, b_ref, hpn_ref):
    dis = dis_ref[...]
    pre = dis * (s2_ref[0] + s2_ref[1] + hp_ref[...]) + b_ref[...]
    act = jnp.maximum(pre, 0.0)
    hpn_ref[...] = jnp.dot(act, w_ref[...], preferred_element_type=jnp.float32) * dis


_mid_tc = pl.pallas_call(
    _mid_body,
    grid=(_GRID,),
    in_specs=[
        pl.BlockSpec((_NC, _RB, _D), lambda i: (0, i, 0)),
        pl.BlockSpec((_RB, _D), lambda i: (i, 0)),
        pl.BlockSpec((_RB, 1), lambda i: (i, 0)),
        pl.BlockSpec((_D, _D), lambda i: (0, 0)),
        pl.BlockSpec((1, _D), lambda i: (0, 0)),
    ],
    out_specs=pl.BlockSpec((_RB, _D), lambda i: (i, 0)),
    out_shape=jax.ShapeDtypeStruct((_NPAD, _D), jnp.float32),
)


def _final_body(s2_ref, hp_ref, dis_ref, b_ref, out_ref):
    dis = dis_ref[...]
    pre = dis * (s2_ref[0] + s2_ref[1] + hp_ref[...]) + b_ref[...]
    out_ref[...] = jnp.maximum(pre, 0.0)


_final_tc = pl.pallas_call(
    _final_body,
    grid=(_GRID,),
    in_specs=[
        pl.BlockSpec((_NC, _RB, _D), lambda i: (0, i, 0)),
        pl.BlockSpec((_RB, _D), lambda i: (i, 0)),
        pl.BlockSpec((_RB, 1), lambda i: (i, 0)),
        pl.BlockSpec((1, _D), lambda i: (0, 0)),
    ],
    out_specs=pl.BlockSpec((_RB, _D), lambda i: (i, 0)),
    out_shape=jax.ShapeDtypeStruct((_NPAD, _D), jnp.float32),
)


def kernel(x, edge_index, W0, b0, W1, b1, W2, b2):
    pad = jnp.full((_EPAD - _E,), _N, jnp.int32)
    src3 = jnp.concatenate([edge_index[0], pad]).reshape(_NW, _NCH, _CB)
    dst3 = jnp.concatenate([edge_index[1], pad]).reshape(_NW, _NCH, _CB)
    xp = jnp.pad(x, ((0, _NPAD - _N), (0, 0)))
    onesD = jnp.ones((_CB, _D), jnp.float32)
    zerosD = jnp.zeros((_RPS, _D), jnp.float32)
    b0r = b0.reshape(1, _D)
    b1r = b1.reshape(1, _D)
    b2r = b2.reshape(1, _D)

    deg16 = _deg_sc(dst3, onesD, zerosD)
    deg2 = jnp.transpose(deg16[:, :, 0])
    dis, hp = _prelude_tc(deg2, xp, W0)

    s = _agg_sc(hp, src3, dst3, zerosD)
    hp = _mid_tc(s, hp, dis, W1, b0r)
    s = _agg_sc(hp, src3, dst3, zerosD)
    hp = _mid_tc(s, hp, dis, W2, b1r)
    s = _agg_sc(hp, src3, dst3, zerosD)
    out = _final_tc(s, hp, dis, b2r)
    return out[:_N]


# speedup vs baseline: 17.0435x; 2.7857x over previous
import functools

import jax
import jax.numpy as jnp
from jax import lax
from jax.experimental import pallas as pl
from jax.experimental.pallas import tpu as pltpu
from jax.experimental.pallas import tpu_sc as plsc

_N = 10000
_E = 320000
_D = 128
_NC = 2
_NS = 16
_NW = _NC * _NS
_CB = 128
_NCH = 80
_EPAD = _NW * _NCH * _CB
_IB = 8
_NIB = _NCH // _IB
_NPAD = 10240
_RPS = _NPAD // _NS
_RB = 1024
_GRID = _NPAD // _RB


def _sc_mesh():
    return plsc.VectorSubcoreMesh(
        core_axis_name="c", subcore_axis_name="s", num_cores=_NC, num_subcores=_NS
    )


@functools.partial(
    pl.kernel,
    out_type=jax.ShapeDtypeStruct((_NC, _NPAD, _D), jnp.float32),
    mesh=_sc_mesh(),
    scratch_types=[
        pltpu.VMEM((_NCH, _CB), jnp.int32),
        pltpu.VMEM((_CB, _D), jnp.float32),
        pltpu.VMEM_SHARED((_NPAD, _D), jnp.float32),
    ],
)
def _deg_sc(dst_hbm, ones_hbm, zeros_hbm, out_hbm, dst_v, ones_v, acc_s):
    c = lax.axis_index("c")
    s = lax.axis_index("s")
    wid = s * _NC + c
    pltpu.sync_copy(zeros_hbm, acc_s.at[pl.ds(s * _RPS, _RPS)])
    pltpu.sync_copy(dst_hbm.at[wid], dst_v)
    pltpu.sync_copy(ones_hbm, ones_v)
    plsc.subcore_barrier()

    def body(i, carry):
        pltpu.sync_copy(ones_v, acc_s.at[dst_v.at[i]], add=True)
        return carry

    lax.fori_loop(0, _NCH, body, 0)
    plsc.subcore_barrier()
    pltpu.sync_copy(acc_s.at[pl.ds(s * _RPS, _RPS)],
                    out_hbm.at[c, pl.ds(s * _RPS, _RPS)])


@functools.partial(
    pl.kernel,
    out_type=jax.ShapeDtypeStruct((_NC, _NPAD, _D), jnp.float32),
    mesh=_sc_mesh(),
    scratch_types=[
        pltpu.VMEM((_IB, _CB), jnp.int32),
        pltpu.VMEM((_IB, _CB), jnp.int32),
        pltpu.VMEM((2, _CB, _D), jnp.float32),
        pltpu.VMEM_SHARED((_NPAD, _D), jnp.float32),
        pltpu.SemaphoreType.DMA,
        pltpu.SemaphoreType.DMA,
    ],
)
def _agg_sc(hp_hbm, src_hbm, dst_hbm, zeros_hbm, out_hbm,
            src_v, dst_v, rows_v, acc_s, sem0, sem1):
    c = lax.axis_index("c")
    s = lax.axis_index("s")
    wid = s * _NC + c
    pltpu.sync_copy(zeros_hbm, acc_s.at[pl.ds(s * _RPS, _RPS)])
    plsc.subcore_barrier()

    def batch(ib, carry):
        pltpu.sync_copy(src_hbm.at[wid, pl.ds(ib * _IB, _IB)], src_v)
        pltpu.sync_copy(dst_hbm.at[wid, pl.ds(ib * _IB, _IB)], dst_v)

        def pair(jp, carry2):
            a = 2 * jp
            ga = pltpu.async_copy(hp_hbm.at[src_v.at[a]], rows_v.at[0], sem0)
            gb = pltpu.async_copy(hp_hbm.at[src_v.at[a + 1]], rows_v.at[1], sem1)
            ga.wait()
            pltpu.sync_copy(rows_v.at[0], acc_s.at[dst_v.at[a]], add=True)
            gb.wait()
            pltpu.sync_copy(rows_v.at[1], acc_s.at[dst_v.at[a + 1]], add=True)
            return carry2

        lax.fori_loop(0, _IB // 2, pair, 0)
        return carry

    lax.fori_loop(0, _NIB, batch, 0)

    plsc.subcore_barrier()
    pltpu.sync_copy(acc_s.at[pl.ds(s * _RPS, _RPS)],
                    out_hbm.at[c, pl.ds(s * _RPS, _RPS)])


def _prelude_body(deg_ref, x_ref, w_ref, dis_ref, hp_ref):
    deg = deg_ref[:, 0:1] + deg_ref[:, 1:2] + 1.0
    dis = lax.rsqrt(deg)
    dis_ref[...] = dis
    h = jnp.dot(x_ref[...], w_ref[...], preferred_element_type=jnp.float32)
    hp_ref[...] = h * dis


_prelude_tc = pl.pallas_call(
    _prelude_body,
    grid=(_GRID,),
    in_specs=[
        pl.BlockSpec((_RB, 2), lambda i: (i, 0)),
        pl.BlockSpec((_RB, _D), lambda i: (i, 0)),
        pl.BlockSpec((_D, _D), lambda i: (0, 0)),
    ],
    out_specs=[
        pl.BlockSpec((_RB, 1), lambda i: (i, 0)),
        pl.BlockSpec((_RB, _D), lambda i: (i, 0)),
    ],
    out_shape=[
        jax.ShapeDtypeStruct((_NPAD, 1), jnp.float32),
        jax.ShapeDtypeStruct((_NPAD, _D), jnp.float32),
    ],
)


def _mid_body(s2_ref, hp_ref, dis_ref, w_ref, b_ref, hpn_ref):
    dis = dis_ref[...]
    pre = dis * (s2_ref[0] + s2_ref[1] + hp_ref[...]) + b_ref[...]
    act = jnp.maximum(pre, 0.0)
    hpn_ref[...] = jnp.dot(act, w_ref[...], preferred_element_type=jnp.float32) * dis


_mid_tc = pl.pallas_call(
    _mid_body,
    grid=(_GRID,),
    in_specs=[
        pl.BlockSpec((_NC, _RB, _D), lambda i: (0, i, 0)),
        pl.BlockSpec((_RB, _D), lambda i: (i, 0)),
        pl.BlockSpec((_RB, 1), lambda i: (i, 0)),
        pl.BlockSpec((_D, _D), lambda i: (0, 0)),
        pl.BlockSpec((1, _D), lambda i: (0, 0)),
    ],
    out_specs=pl.BlockSpec((_RB, _D), lambda i: (i, 0)),
    out_shape=jax.ShapeDtypeStruct((_NPAD, _D), jnp.float32),
)


def _final_body(s2_ref, hp_ref, dis_ref, b_ref, out_ref):
    dis = dis_ref[...]
    pre = dis * (s2_ref[0] + s2_ref[1] + hp_ref[...]) + b_ref[...]
    out_ref[...] = jnp.maximum(pre, 0.0)


_final_tc = pl.pallas_call(
    _final_body,
    grid=(_GRID,),
    in_specs=[
        pl.BlockSpec((_NC, _RB, _D), lambda i: (0, i, 0)),
        pl.BlockSpec((_RB, _D), lambda i: (i, 0)),
        pl.BlockSpec((_RB, 1), lambda i: (i, 0)),
        pl.BlockSpec((1, _D), lambda i: (0, 0)),
    ],
    out_specs=pl.BlockSpec((_RB, _D), lambda i: (i, 0)),
    out_shape=jax.ShapeDtypeStruct((_NPAD, _D), jnp.float32),
)


def kernel(x, edge_index, W0, b0, W1, b1, W2, b2):
    pad = _N + jnp.arange(_EPAD - _E, dtype=jnp.int32) % (_NPAD - _N)
    src3 = jnp.concatenate([edge_index[0], pad]).reshape(_NW, _NCH, _CB)
    dst3 = jnp.concatenate([edge_index[1], pad]).reshape(_NW, _NCH, _CB)
    xp = jnp.pad(x, ((0, _NPAD - _N), (0, 0)))
    onesD = jnp.ones((_CB, _D), jnp.float32)
    zerosD = jnp.zeros((_RPS, _D), jnp.float32)
    b0r = b0.reshape(1, _D)
    b1r = b1.reshape(1, _D)
    b2r = b2.reshape(1, _D)

    deg16 = _deg_sc(dst3, onesD, zerosD)
    deg2 = jnp.transpose(deg16[:, :, 0])
    dis, hp = _prelude_tc(deg2, xp, W0)

    s = _agg_sc(hp, src3, dst3, zerosD)
    hp = _mid_tc(s, hp, dis, W1, b0r)
    s = _agg_sc(hp, src3, dst3, zerosD)
    hp = _mid_tc(s, hp, dis, W2, b1r)
    s = _agg_sc(hp, src3, dst3, zerosD)
    out = _final_tc(s, hp, dis, b2r)
    return out[:_N]


# speedup vs baseline: 24.6759x; 1.4478x over previous
import functools

import jax
import jax.numpy as jnp
from jax import lax
from jax.experimental import pallas as pl
from jax.experimental.pallas import tpu as pltpu
from jax.experimental.pallas import tpu_sc as plsc

_N = 10000
_E = 320000
_D = 128
_NC = 2
_NS = 16
_NW = _NC * _NS
_CB = 128
_NCH = 80
_EPAD = _NW * _NCH * _CB
_IB = 16
_NIB = _NCH // _IB
_NPAD = 10240
_RPS = _NPAD // _NS
_RB = 1000
_GRID = _N // _RB


def _sc_mesh():
    return plsc.VectorSubcoreMesh(
        core_axis_name="c", subcore_axis_name="s", num_cores=_NC, num_subcores=_NS
    )


@functools.partial(
    pl.kernel,
    out_type=jax.ShapeDtypeStruct((_NC, _NPAD, _D), jnp.float32),
    mesh=_sc_mesh(),
    scratch_types=[
        pltpu.VMEM((_NCH, _CB), jnp.int32),
        pltpu.VMEM((_CB, _D), jnp.float32),
        pltpu.VMEM_SHARED((_NPAD, _D), jnp.float32),
    ],
)
def _deg_sc(dst_hbm, ones_hbm, zeros_hbm, out_hbm, dst_v, ones_v, acc_s):
    c = lax.axis_index("c")
    s = lax.axis_index("s")
    wid = s * _NC + c
    pltpu.sync_copy(zeros_hbm, acc_s.at[pl.ds(s * _RPS, _RPS)])
    pltpu.sync_copy(dst_hbm.at[wid], dst_v)
    pltpu.sync_copy(ones_hbm, ones_v)
    plsc.subcore_barrier()

    def body(i, carry):
        pltpu.sync_copy(ones_v, acc_s.at[dst_v.at[i]], add=True)
        return carry

    lax.fori_loop(0, _NCH, body, 0)
    plsc.subcore_barrier()
    pltpu.sync_copy(acc_s.at[pl.ds(s * _RPS, _RPS)],
                    out_hbm.at[c, pl.ds(s * _RPS, _RPS)])


@functools.partial(
    pl.kernel,
    out_type=jax.ShapeDtypeStruct((_NC, _NPAD, _D), jnp.float32),
    mesh=_sc_mesh(),
    scratch_types=[
        pltpu.VMEM((_NCH, _CB), jnp.int32),
        pltpu.VMEM((2, _IB, _CB), jnp.int32),
        pltpu.VMEM((2, _CB, _D), jnp.float32),
        pltpu.VMEM_SHARED((_NPAD, _D), jnp.float32),
        pltpu.SemaphoreType.DMA,
        pltpu.SemaphoreType.DMA,
        pltpu.SemaphoreType.DMA,
    ],
)
def _agg_sc(hp_hbm, src_hbm, dst_hbm, zeros_hbm, out_hbm,
            src_v, dst_v, rows_v, acc_s, sem0, sem1, semi):
    c = lax.axis_index("c")
    s = lax.axis_index("s")
    wid = s * _NC + c
    pltpu.sync_copy(zeros_hbm, acc_s.at[pl.ds(s * _RPS, _RPS)])
    pltpu.sync_copy(src_hbm.at[wid], src_v)
    pltpu.sync_copy(dst_hbm.at[wid, pl.ds(0, _IB)], dst_v.at[0])
    plsc.subcore_barrier()

    sems = (sem0, sem1)

    def gather(i, buf):
        return pltpu.async_copy(hp_hbm.at[src_v.at[i]], rows_v.at[buf], sems[buf])

    gather(0, 0)
    gather(1, 1)

    for ib in range(_NIB):
        slot = ib % 2
        if ib + 1 < _NIB:
            nxt = pltpu.async_copy(
                dst_hbm.at[wid, pl.ds((ib + 1) * _IB, _IB)],
                dst_v.at[1 - slot], semi)
        base = ib * _IB

        def pair(jp, carry, base=base, slot=slot):
            a = base + 2 * jp
            for k in (0, 1):
                pltpu.make_async_copy(
                    hp_hbm.at[src_v.at[a + k]], rows_v.at[k], sems[k]).wait()
                pltpu.sync_copy(rows_v.at[k],
                                acc_s.at[dst_v.at[slot, 2 * jp + k]], add=True)

                @pl.when(a + k + 2 < _NCH)
                def _(a=a, k=k):
                    gather(a + k + 2, k)
            return carry

        lax.fori_loop(0, _IB // 2, pair, 0)
        if ib + 1 < _NIB:
            nxt.wait()

    plsc.subcore_barrier()
    pltpu.sync_copy(acc_s.at[pl.ds(s * _RPS, _RPS)],
                    out_hbm.at[c, pl.ds(s * _RPS, _RPS)])


def _prelude_body(deg_ref, x_ref, w_ref, dis_ref, hp_ref):
    deg = deg_ref[0, :, 0:1] + deg_ref[1, :, 0:1] + 1.0
    dis = lax.rsqrt(deg)
    dis_ref[...] = dis
    h = jnp.dot(x_ref[...], w_ref[...], preferred_element_type=jnp.float32)
    hp_ref[...] = h * dis


_prelude_tc = pl.pallas_call(
    _prelude_body,
    grid=(_GRID,),
    in_specs=[
        pl.BlockSpec((_NC, _RB, _D), lambda i: (0, i, 0)),
        pl.BlockSpec((_RB, _D), lambda i: (i, 0)),
        pl.BlockSpec((_D, _D), lambda i: (0, 0)),
    ],
    out_specs=[
        pl.BlockSpec((_RB, 1), lambda i: (i, 0)),
        pl.BlockSpec((_RB, _D), lambda i: (i, 0)),
    ],
    out_shape=[
        jax.ShapeDtypeStruct((_N, 1), jnp.float32),
        jax.ShapeDtypeStruct((_N, _D), jnp.float32),
    ],
)


def _mid_body(s2_ref, hp_ref, dis_ref, w_ref, b_ref, hpn_ref):
    dis = dis_ref[...]
    pre = dis * (s2_ref[0] + s2_ref[1] + hp_ref[...]) + b_ref[...]
    act = jnp.maximum(pre, 0.0)
    hpn_ref[...] = jnp.dot(act, w_ref[...], preferred_element_type=jnp.float32) * dis


_mid_tc = pl.pallas_call(
    _mid_body,
    grid=(_GRID,),
    in_specs=[
        pl.BlockSpec((_NC, _RB, _D), lambda i: (0, i, 0)),
        pl.BlockSpec((_RB, _D), lambda i: (i, 0)),
        pl.BlockSpec((_RB, 1), lambda i: (i, 0)),
        pl.BlockSpec((_D, _D), lambda i: (0, 0)),
        pl.BlockSpec((1, _D), lambda i: (0, 0)),
    ],
    out_specs=pl.BlockSpec((_RB, _D), lambda i: (i, 0)),
    out_shape=jax.ShapeDtypeStruct((_N, _D), jnp.float32),
)


def _final_body(s2_ref, hp_ref, dis_ref, b_ref, out_ref):
    dis = dis_ref[...]
    pre = dis * (s2_ref[0] + s2_ref[1] + hp_ref[...]) + b_ref[...]
    out_ref[...] = jnp.maximum(pre, 0.0)


_final_tc = pl.pallas_call(
    _final_body,
    grid=(_GRID,),
    in_specs=[
        pl.BlockSpec((_NC, _RB, _D), lambda i: (0, i, 0)),
        pl.BlockSpec((_RB, _D), lambda i: (i, 0)),
        pl.BlockSpec((_RB, 1), lambda i: (i, 0)),
        pl.BlockSpec((1, _D), lambda i: (0, 0)),
    ],
    out_specs=pl.BlockSpec((_RB, _D), lambda i: (i, 0)),
    out_shape=jax.ShapeDtypeStruct((_N, _D), jnp.float32),
)


def kernel(x, edge_index, W0, b0, W1, b1, W2, b2):
    npd = _NPAD - _N
    idx = jnp.arange(_EPAD - _E, dtype=jnp.int32)
    src3 = jnp.concatenate([edge_index[0], idx % npd]).reshape(_NW, _NCH, _CB)
    dst3 = jnp.concatenate([edge_index[1], _N + idx % npd]).reshape(_NW, _NCH, _CB)
    onesD = jnp.ones((_CB, _D), jnp.float32)
    zerosD = jnp.zeros((_RPS, _D), jnp.float32)
    b0r = b0.reshape(1, _D)
    b1r = b1.reshape(1, _D)
    b2r = b2.reshape(1, _D)

    deg = _deg_sc(dst3, onesD, zerosD)
    dis, hp = _prelude_tc(deg, x, W0)

    s = _agg_sc(hp, src3, dst3, zerosD)
    hp = _mid_tc(s, hp, dis, W1, b0r)
    s = _agg_sc(hp, src3, dst3, zerosD)
    hp = _mid_tc(s, hp, dis, W2, b1r)
    s = _agg_sc(hp, src3, dst3, zerosD)
    out = _final_tc(s, hp, dis, b2r)
    return out


# speedup vs baseline: 24.9160x; 1.0097x over previous
import functools

import jax
import jax.numpy as jnp
from jax import lax
from jax.experimental import pallas as pl
from jax.experimental.pallas import tpu as pltpu
from jax.experimental.pallas import tpu_sc as plsc

_N = 10000
_E = 320000
_D = 128
_NC = 2
_NS = 16
_NW = _NC * _NS
_CB = 128
_NCH = 80
_EPAD = _NW * _NCH * _CB
_IB = 16
_NIB = _NCH // _IB
_NPAD = 10240
_RPS = _NPAD // _NS
_RB = 1000
_GRID = _N // _RB


def _sc_mesh():
    return plsc.VectorSubcoreMesh(
        core_axis_name="c", subcore_axis_name="s", num_cores=_NC, num_subcores=_NS
    )


@functools.partial(
    pl.kernel,
    out_type=jax.ShapeDtypeStruct((_NC, _NPAD, _D), jnp.float32),
    mesh=_sc_mesh(),
    scratch_types=[
        pltpu.VMEM((_NCH, _CB), jnp.int32),
        pltpu.VMEM((_CB, _D), jnp.float32),
        pltpu.VMEM_SHARED((_NPAD, _D), jnp.float32),
    ],
)
def _deg_sc(dst_hbm, ones_hbm, zeros_hbm, out_hbm, dst_v, ones_v, acc_s):
    c = lax.axis_index("c")
    s = lax.axis_index("s")
    wid = s * _NC + c
    pltpu.sync_copy(zeros_hbm, acc_s.at[pl.ds(s * _RPS, _RPS)])
    pltpu.sync_copy(dst_hbm.at[wid], dst_v)
    pltpu.sync_copy(ones_hbm, ones_v)
    plsc.subcore_barrier()

    def body(i, carry):
        pltpu.sync_copy(ones_v, acc_s.at[dst_v.at[i]], add=True)
        return carry

    lax.fori_loop(0, _NCH, body, 0)
    plsc.subcore_barrier()
    pltpu.sync_copy(acc_s.at[pl.ds(s * _RPS, _RPS)],
                    out_hbm.at[c, pl.ds(s * _RPS, _RPS)])


@functools.partial(
    pl.kernel,
    out_type=jax.ShapeDtypeStruct((_NC, _NPAD, _D), jnp.float32),
    mesh=_sc_mesh(),
    scratch_types=[
        pltpu.VMEM((_NCH, _CB), jnp.int32),
        pltpu.VMEM((2, _IB, _CB), jnp.int32),
        pltpu.VMEM((2, _CB, _D), jnp.float32),
        pltpu.VMEM_SHARED((_NPAD, _D), jnp.float32),
        pltpu.SemaphoreType.DMA,
        pltpu.SemaphoreType.DMA,
        pltpu.SemaphoreType.DMA,
    ],
)
def _agg_sc(hp_hbm, src_hbm, dst_hbm, zeros_hbm, out_hbm,
            src_v, dst_v, rows_v, acc_s, sem0, sem1, semi):
    c = lax.axis_index("c")
    s = lax.axis_index("s")
    wid = s * _NC + c
    pltpu.sync_copy(zeros_hbm, acc_s.at[pl.ds(s * _RPS, _RPS)])
    pltpu.sync_copy(src_hbm.at[wid], src_v)
    pltpu.sync_copy(dst_hbm.at[wid, pl.ds(0, _IB)], dst_v.at[0])
    plsc.subcore_barrier()

    sems = (sem0, sem1)
    _H = _CB // 2

    def gather(i, buf):
        pltpu.async_copy(hp_hbm.at[src_v.at[i, pl.ds(0, _H)]],
                         rows_v.at[buf, pl.ds(0, _H)], sems[buf])
        pltpu.async_copy(hp_hbm.at[src_v.at[i, pl.ds(_H, _H)]],
                         rows_v.at[buf, pl.ds(_H, _H)], sems[buf])

    gather(0, 0)
    gather(1, 1)

    for ib in range(_NIB):
        slot = ib % 2
        if ib + 1 < _NIB:
            nxt = pltpu.async_copy(
                dst_hbm.at[wid, pl.ds((ib + 1) * _IB, _IB)],
                dst_v.at[1 - slot], semi)
        base = ib * _IB

        def pair(jp, carry, base=base, slot=slot):
            a = base + 2 * jp
            for k in (0, 1):
                pltpu.make_async_copy(
                    hp_hbm.at[src_v.at[a + k, pl.ds(0, _H)]],
                    rows_v.at[k, pl.ds(0, _H)], sems[k]).wait()
                pltpu.make_async_copy(
                    hp_hbm.at[src_v.at[a + k, pl.ds(_H, _H)]],
                    rows_v.at[k, pl.ds(_H, _H)], sems[k]).wait()
                pltpu.sync_copy(rows_v.at[k],
                                acc_s.at[dst_v.at[slot, 2 * jp + k]], add=True)

                @pl.when(a + k + 2 < _NCH)
                def _(a=a, k=k):
                    gather(a + k + 2, k)
            return carry

        lax.fori_loop(0, _IB // 2, pair, 0)
        if ib + 1 < _NIB:
            nxt.wait()

    plsc.subcore_barrier()
    pltpu.sync_copy(acc_s.at[pl.ds(s * _RPS, _RPS)],
                    out_hbm.at[c, pl.ds(s * _RPS, _RPS)])


def _prelude_body(deg_ref, x_ref, w_ref, dis_ref, hp_ref):
    deg = deg_ref[0, :, 0:1] + deg_ref[1, :, 0:1] + 1.0
    dis = lax.rsqrt(deg)
    dis_ref[...] = dis
    h = jnp.dot(x_ref[...], w_ref[...], preferred_element_type=jnp.float32)
    hp_ref[...] = h * dis


_prelude_tc = pl.pallas_call(
    _prelude_body,
    grid=(_GRID,),
    in_specs=[
        pl.BlockSpec((_NC, _RB, _D), lambda i: (0, i, 0)),
        pl.BlockSpec((_RB, _D), lambda i: (i, 0)),
        pl.BlockSpec((_D, _D), lambda i: (0, 0)),
    ],
    out_specs=[
        pl.BlockSpec((_RB, 1), lambda i: (i, 0)),
        pl.BlockSpec((_RB, _D), lambda i: (i, 0)),
    ],
    out_shape=[
        jax.ShapeDtypeStruct((_N, 1), jnp.float32),
        jax.ShapeDtypeStruct((_N, _D), jnp.float32),
    ],
)


def _mid_body(s2_ref, hp_ref, dis_ref, w_ref, b_ref, hpn_ref):
    dis = dis_ref[...]
    pre = dis * (s2_ref[0] + s2_ref[1] + hp_ref[...]) + b_ref[...]
    act = jnp.maximum(pre, 0.0)
    hpn_ref[...] = jnp.dot(act, w_ref[...], preferred_element_type=jnp.float32) * dis


_mid_tc = pl.pallas_call(
    _mid_body,
    grid=(_GRID,),
    in_specs=[
        pl.BlockSpec((_NC, _RB, _D), lambda i: (0, i, 0)),
        pl.BlockSpec((_RB, _D), lambda i: (i, 0)),
        pl.BlockSpec((_RB, 1), lambda i: (i, 0)),
        pl.BlockSpec((_D, _D), lambda i: (0, 0)),
        pl.BlockSpec((1, _D), lambda i: (0, 0)),
    ],
    out_specs=pl.BlockSpec((_RB, _D), lambda i: (i, 0)),
    out_shape=jax.ShapeDtypeStruct((_N, _D), jnp.float32),
)


def _final_body(s2_ref, hp_ref, dis_ref, b_ref, out_ref):
    dis = dis_ref[...]
    pre = dis * (s2_ref[0] + s2_ref[1] + hp_ref[...]) + b_ref[...]
    out_ref[...] = jnp.maximum(pre, 0.0)


_final_tc = pl.pallas_call(
    _final_body,
    grid=(_GRID,),
    in_specs=[
        pl.BlockSpec((_NC, _RB, _D), lambda i: (0, i, 0)),
        pl.BlockSpec((_RB, _D), lambda i: (i, 0)),
        pl.BlockSpec((_RB, 1), lambda i: (i, 0)),
        pl.BlockSpec((1, _D), lambda i: (0, 0)),
    ],
    out_specs=pl.BlockSpec((_RB, _D), lambda i: (i, 0)),
    out_shape=jax.ShapeDtypeStruct((_N, _D), jnp.float32),
)


def kernel(x, edge_index, W0, b0, W1, b1, W2, b2):
    npd = _NPAD - _N
    idx = jnp.arange(_EPAD - _E, dtype=jnp.int32)
    src3 = jnp.concatenate([edge_index[0], idx % npd]).reshape(_NW, _NCH, _CB)
    dst3 = jnp.concatenate([edge_index[1], _N + idx % npd]).reshape(_NW, _NCH, _CB)
    onesD = jnp.ones((_CB, _D), jnp.float32)
    zerosD = jnp.zeros((_RPS, _D), jnp.float32)
    b0r = b0.reshape(1, _D)
    b1r = b1.reshape(1, _D)
    b2r = b2.reshape(1, _D)

    deg = _deg_sc(dst3, onesD, zerosD)
    dis, hp = _prelude_tc(deg, x, W0)

    s = _agg_sc(hp, src3, dst3, zerosD)
    hp = _mid_tc(s, hp, dis, W1, b0r)
    s = _agg_sc(hp, src3, dst3, zerosD)
    hp = _mid_tc(s, hp, dis, W2, b1r)
    s = _agg_sc(hp, src3, dst3, zerosD)
    out = _final_tc(s, hp, dis, b2r)
    return out


# speedup vs baseline: 24.9981x; 1.0033x over previous
import functools

import numpy as np

import jax
import jax.numpy as jnp
from jax import lax
from jax.experimental import pallas as pl
from jax.experimental.pallas import tpu as pltpu
from jax.experimental.pallas import tpu_sc as plsc

_N = 10000
_E = 320000
_D = 128
_NC = 2
_NS = 16
_NW = _NC * _NS
_CB = 128
_NCH = 80
_EPAD = _NW * _NCH * _CB
_IB = 16
_NIB = _NCH // _IB
_NPAD = 10240
_RPS = _NPAD // _NS
_RB = 1000
_GRID = _N // _RB


def _sc_mesh():
    return plsc.VectorSubcoreMesh(
        core_axis_name="c", subcore_axis_name="s", num_cores=_NC, num_subcores=_NS
    )


@functools.partial(
    pl.kernel,
    out_type=jax.ShapeDtypeStruct((_NC, _NPAD, _D), jnp.float32),
    mesh=_sc_mesh(),
    scratch_types=[
        pltpu.VMEM((_NCH, _CB), jnp.int32),
        pltpu.VMEM((_CB, _D), jnp.float32),
        pltpu.VMEM_SHARED((_NPAD, _D), jnp.float32),
    ],
)
def _deg_sc(dst_hbm, ones_hbm, zeros_hbm, out_hbm, dst_v, ones_v, acc_s):
    c = lax.axis_index("c")
    s = lax.axis_index("s")
    wid = s * _NC + c
    pltpu.sync_copy(zeros_hbm, acc_s.at[pl.ds(s * _RPS, _RPS)])
    pltpu.sync_copy(dst_hbm.at[wid], dst_v)
    pltpu.sync_copy(ones_hbm, ones_v)
    plsc.subcore_barrier()

    def body(i, carry):
        pltpu.sync_copy(ones_v, acc_s.at[dst_v.at[i]], add=True)
        return carry

    lax.fori_loop(0, _NCH, body, 0)
    plsc.subcore_barrier()
    pltpu.sync_copy(acc_s.at[pl.ds(s * _RPS, _RPS)],
                    out_hbm.at[c, pl.ds(s * _RPS, _RPS)])


@functools.partial(
    pl.kernel,
    out_type=jax.ShapeDtypeStruct((_NC, _NPAD, _D), jnp.float32),
    mesh=_sc_mesh(),
    scratch_types=[
        pltpu.VMEM((_NCH, _CB), jnp.int32),
        pltpu.VMEM((2, _IB, _CB), jnp.int32),
        pltpu.VMEM((2, _CB, _D), jnp.float32),
        pltpu.VMEM_SHARED((_NPAD, _D), jnp.float32),
        pltpu.SemaphoreType.DMA,
        pltpu.SemaphoreType.DMA,
        pltpu.SemaphoreType.DMA,
    ],
)
def _agg_sc(hp_hbm, src_hbm, dst_hbm, zeros_hbm, out_hbm,
            src_v, dst_v, rows_v, acc_s, sem0, sem1, semi):
    c = lax.axis_index("c")
    s = lax.axis_index("s")
    wid = s * _NC + c
    pltpu.sync_copy(zeros_hbm, acc_s.at[pl.ds(s * _RPS, _RPS)])
    pltpu.sync_copy(src_hbm.at[wid], src_v)
    pltpu.sync_copy(dst_hbm.at[wid, pl.ds(0, _IB)], dst_v.at[0])
    plsc.subcore_barrier()

    sems = (sem0, sem1)
    _H = _CB // 2

    def gather(i, buf):
        pltpu.async_copy(hp_hbm.at[src_v.at[i, pl.ds(0, _H)]],
                         rows_v.at[buf, pl.ds(0, _H)], sems[buf])
        pltpu.async_copy(hp_hbm.at[src_v.at[i, pl.ds(_H, _H)]],
                         rows_v.at[buf, pl.ds(_H, _H)], sems[buf])

    gather(0, 0)
    gather(1, 1)

    for ib in range(_NIB):
        slot = ib % 2
        if ib + 1 < _NIB:
            nxt = pltpu.async_copy(
                dst_hbm.at[wid, pl.ds((ib + 1) * _IB, _IB)],
                dst_v.at[1 - slot], semi)
        base = ib * _IB

        def pair(jp, carry, base=base, slot=slot):
            a = base + 2 * jp
            for k in (0, 1):
                pltpu.make_async_copy(
                    hp_hbm.at[src_v.at[a + k, pl.ds(0, _H)]],
                    rows_v.at[k, pl.ds(0, _H)], sems[k]).wait()
                pltpu.make_async_copy(
                    hp_hbm.at[src_v.at[a + k, pl.ds(_H, _H)]],
                    rows_v.at[k, pl.ds(_H, _H)], sems[k]).wait()
                pltpu.sync_copy(rows_v.at[k],
                                acc_s.at[dst_v.at[slot, 2 * jp + k]], add=True)

                @pl.when(a + k + 2 < _NCH)
                def _(a=a, k=k):
                    gather(a + k + 2, k)
            return carry

        lax.fori_loop(0, _IB // 2, pair, 0)
        if ib + 1 < _NIB:
            nxt.wait()

    plsc.subcore_barrier()
    pltpu.sync_copy(acc_s.at[pl.ds(s * _RPS, _RPS)],
                    out_hbm.at[c, pl.ds(s * _RPS, _RPS)])


def _prelude_body(deg_ref, x_ref, w_ref, dis_ref, hp_ref):
    deg = deg_ref[0, :, 0:1] + deg_ref[1, :, 0:1] + 1.0
    dis = lax.rsqrt(deg)
    dis_ref[...] = dis
    h = jnp.dot(x_ref[...], w_ref[...], preferred_element_type=jnp.float32)
    hp_ref[...] = h * dis


_prelude_tc = pl.pallas_call(
    _prelude_body,
    grid=(_GRID,),
    in_specs=[
        pl.BlockSpec((_NC, _RB, 16), lambda i: (0, i, 0)),
        pl.BlockSpec((_RB, _D), lambda i: (i, 0)),
        pl.BlockSpec((_D, _D), lambda i: (0, 0)),
    ],
    out_specs=[
        pl.BlockSpec((_RB, 1), lambda i: (i, 0)),
        pl.BlockSpec((_RB, _D), lambda i: (i, 0)),
    ],
    out_shape=[
        jax.ShapeDtypeStruct((_N, 1), jnp.float32),
        jax.ShapeDtypeStruct((_N, _D), jnp.float32),
    ],
)


def _mid_body(s2_ref, hp_ref, dis_ref, w_ref, b_ref, hpn_ref):
    dis = dis_ref[...]
    pre = dis * (s2_ref[0] + s2_ref[1] + hp_ref[...]) + b_ref[...]
    act = jnp.maximum(pre, 0.0)
    hpn_ref[...] = jnp.dot(act, w_ref[...], preferred_element_type=jnp.float32) * dis


_mid_tc = pl.pallas_call(
    _mid_body,
    grid=(_GRID,),
    in_specs=[
        pl.BlockSpec((_NC, _RB, _D), lambda i: (0, i, 0)),
        pl.BlockSpec((_RB, _D), lambda i: (i, 0)),
        pl.BlockSpec((_RB, 1), lambda i: (i, 0)),
        pl.BlockSpec((_D, _D), lambda i: (0, 0)),
        pl.BlockSpec((1, _D), lambda i: (0, 0)),
    ],
    out_specs=pl.BlockSpec((_RB, _D), lambda i: (i, 0)),
    out_shape=jax.ShapeDtypeStruct((_N, _D), jnp.float32),
)


def _final_body(s2_ref, hp_ref, dis_ref, b_ref, out_ref):
    dis = dis_ref[...]
    pre = dis * (s2_ref[0] + s2_ref[1] + hp_ref[...]) + b_ref[...]
    out_ref[...] = jnp.maximum(pre, 0.0)


_final_tc = pl.pallas_call(
    _final_body,
    grid=(_GRID,),
    in_specs=[
        pl.BlockSpec((_NC, _RB, _D), lambda i: (0, i, 0)),
        pl.BlockSpec((_RB, _D), lambda i: (i, 0)),
        pl.BlockSpec((_RB, 1), lambda i: (i, 0)),
        pl.BlockSpec((1, _D), lambda i: (0, 0)),
    ],
    out_specs=pl.BlockSpec((_RB, _D), lambda i: (i, 0)),
    out_shape=jax.ShapeDtypeStruct((_N, _D), jnp.float32),
)


def kernel(x, edge_index, W0, b0, W1, b1, W2, b2):
    npd = _NPAD - _N
    idx = np.arange(_EPAD - _E, dtype=np.int32) % npd
    src3 = jnp.concatenate([edge_index[0], jnp.asarray(idx)]).reshape(_NW, _NCH, _CB)
    dst3 = jnp.concatenate([edge_index[1], jnp.asarray(_N + idx)]).reshape(_NW, _NCH, _CB)
    onesD = jnp.ones((_CB, _D), jnp.float32)
    zerosD = jnp.zeros((_RPS, _D), jnp.float32)
    b0r = b0.reshape(1, _D)
    b1r = b1.reshape(1, _D)
    b2r = b2.reshape(1, _D)

    deg = _deg_sc(dst3, onesD, zerosD)
    dis, hp = _prelude_tc(deg[:, :, :16], x, W0)

    s = _agg_sc(hp, src3, dst3, zerosD)
    hp = _mid_tc(s, hp, dis, W1, b0r)
    s = _agg_sc(hp, src3, dst3, zerosD)
    hp = _mid_tc(s, hp, dis, W2, b1r)
    s = _agg_sc(hp, src3, dst3, zerosD)
    out = _final_tc(s, hp, dis, b2r)
    return out
